# baseline (speedup 1.0000x reference)
import functools
import jax
import jax.numpy as jnp
from jax import lax
from jax.experimental import pallas as pl
from jax.experimental.pallas import tpu as pltpu
from jax.experimental.pallas import tpu_sc as plsc

EMB = 1024
NPOOL = 8192
TOPK = 8
NBLK = 8
ROWS = NPOOL // NBLK

BATCH = 4096
BBLK = 256
NB = BATCH // BBLK

LANES = 16
NCHUNK = NPOOL // LANES
NGRP = 32
CPG = NCHUNK // NGRP
BIGI = jnp.int32(2 ** 30)


def _sims_kernel(q_ref, keys_ref, s_ref):
    i = pl.program_id(0)
    kb = keys_ref[...]
    q = q_ref[...]
    kq = jnp.dot(kb.astype(jnp.bfloat16), q.T.astype(jnp.bfloat16),
                 preferred_element_type=jnp.float32)
    kn = jnp.sqrt(jnp.sum(kb * kb, axis=1, keepdims=True))
    qn = jnp.sqrt(jnp.sum(q * q))
    sims = kq[:, 0] / jnp.maximum(kn[:, 0] * qn, 1e-8)
    s_ref[0, 0, :] = sims.reshape(1, ROWS)[0, :]


GRPSZ = CPG * LANES


def _sc_topk_gather(sims_hbm, prompts_hbm, out_hbm,
                    sims_v, gv_v, gi_v, gb_v, idx_v, rows_v, sem):
    cid = lax.axis_index("c")
    sid = lax.axis_index("s")

    pltpu.sync_copy(sims_hbm, sims_v)
    lane = lax.iota(jnp.int32, LANES)
    neg = jnp.full((LANES,), -jnp.inf, jnp.float32)
    zero_i = jnp.zeros((LANES,), jnp.int32)
    idxvec = zero_i

    dnums = jax.lax.GatherDimensionNumbers(
        offset_dims=(), collapsed_slice_dims=(0,), start_index_map=(0,))

    def _take16(v, perm):
        return jax.lax.gather(
            v, perm[:, None], dimension_numbers=dnums, slice_sizes=(1,),
            mode=jax.lax.GatherScatterMode.PROMISE_IN_BOUNDS)

    def _pair_fold(av, ai, bv, bi):
        t = (bv > av) | ((bv == av) & (bi < ai))
        return jnp.where(t, bv, av), jnp.where(t, bi, ai)

    def _lane_fold(v, i):
        for s in (8, 4, 2, 1):
            perm = (lane + s) & (LANES - 1)
            v, i = _pair_fold(v, i, _take16(v, perm), _take16(i, perm))
        return v, i

    def _build_group(base):
        bv, bi = neg, zero_i
        for c in range(CPG):
            off = (base * CPG + c) * LANES
            ch = sims_v[pl.ds(off, LANES)]
            gidx = off + lane
            t = ch > bv
            bv = jnp.where(t, ch, bv)
            bi = jnp.where(t, gidx, bi)
        return bv, bi

    for g in range(NGRP):
        bv, bi = _build_group(g)
        gv_v[pl.ds(g * LANES, LANES)] = bv
        gi_v[pl.ds(g * LANES, LANES)] = bi

    def _top_winner():
        tv, ti = neg, zero_i
        for g in range(NGRP):
            tv, ti = _pair_fold(tv, ti,
                                gv_v[pl.ds(g * LANES, LANES)],
                                gi_v[pl.ds(g * LANES, LANES)])
        return _lane_fold(tv, ti)

    for k in range(TOPK):
        _, wi = _top_winner()
        idxvec = jnp.where(lane == k, wi, idxvec)
        gb_v[...] = jax.lax.shift_right_logical(wi, 4)
        c_s = gb_v[...][0]
        ch = sims_v[pl.ds(c_s * LANES, LANES)]
        ch = jnp.where(lane == (wi & (LANES - 1)), neg, ch)
        sims_v[pl.ds(c_s * LANES, LANES)] = ch
        g_s = jax.lax.shift_right_logical(c_s, 4)
        bv, bi = _build_group(g_s)
        gv_v[pl.ds(g_s * LANES, LANES)] = bv
        gi_v[pl.ds(g_s * LANES, LANES)] = bi

    idx_v[...] = idxvec

    @pl.when((cid == 0) & (sid == 0))
    def _():
        pltpu.async_copy(prompts_hbm.at[idx_v], rows_v, sem).wait()
        pltpu.sync_copy(rows_v.at[pl.ds(0, TOPK)], out_hbm)


def _add_kernel(sel_ref, x_ref, o_ref):
    o_ref[...] = x_ref[...] + sel_ref[...][None, :, :]


@jax.jit
def kernel(x, query, prompts, keys):
    q2 = query.reshape(1, EMB)
    sims = pl.pallas_call(
        _sims_kernel,
        grid=(NBLK,),
        in_specs=[
            pl.BlockSpec((1, EMB), lambda i: (0, 0)),
            pl.BlockSpec((ROWS, EMB), lambda i: (i, 0)),
        ],
        out_specs=pl.BlockSpec((1, 1, ROWS), lambda i: (i, 0, 0)),
        out_shape=jax.ShapeDtypeStruct((NBLK, 1, ROWS), jnp.float32),
    )(q2, keys)

    sc_fn = functools.partial(
        pl.kernel,
        mesh=plsc.VectorSubcoreMesh(core_axis_name="c", subcore_axis_name="s"),
        out_type=jax.ShapeDtypeStruct((TOPK, EMB), jnp.float32),
        scratch_types=[
            pltpu.VMEM((NPOOL,), jnp.float32),
            pltpu.VMEM((NGRP * LANES,), jnp.float32),
            pltpu.VMEM((NGRP * LANES,), jnp.int32),
            pltpu.VMEM((LANES,), jnp.int32),
            pltpu.VMEM((LANES,), jnp.int32),
            pltpu.VMEM((LANES, EMB), jnp.float32),
            pltpu.SemaphoreType.DMA,
        ],
    )(_sc_topk_gather)
    selected = sc_fn(sims.reshape(NPOOL), prompts)

    out = pl.pallas_call(
        _add_kernel,
        grid=(NB,),
        in_specs=[
            pl.BlockSpec((TOPK, EMB), lambda b: (0, 0)),
            pl.BlockSpec((BBLK, TOPK, EMB), lambda b: (b, 0, 0)),
        ],
        out_specs=pl.BlockSpec((BBLK, TOPK, EMB), lambda b: (b, 0, 0)),
        out_shape=jax.ShapeDtypeStruct((BATCH, TOPK, EMB), jnp.float32),
    )(selected, x)
    return out


# speedup vs baseline: 1.2960x; 1.2960x over previous
import jax
import jax.numpy as jnp
from jax import lax
from jax.experimental import pallas as pl
from jax.experimental.pallas import tpu as pltpu

EMB = 1024
NPOOL = 8192
TOPK = 8
NBLK = 8
ROWS = NPOOL // NBLK

BATCH = 4096
BBLK = 128
NB = BATCH // BBLK
RING = NBLK + 1


def _fused_kernel(q_ref, keys_ref, x_hbm, prompts_hbm, o_ref,
                  sims_ref, sel_ref, x_bufs, gsem, xsems):
    i = pl.program_id(0)

    @pl.when(i < NB)
    def _issue():
        slot = lax.rem(i, RING)
        pltpu.make_async_copy(
            x_hbm.at[pl.ds(i * BBLK, BBLK)],
            x_bufs.at[slot],
            xsems.at[slot],
        ).start()

    @pl.when(i < NBLK)
    def _sims():
        kb = keys_ref[...]
        q = q_ref[...]
        kq = jnp.dot(kb.astype(jnp.bfloat16), q.T.astype(jnp.bfloat16),
                     preferred_element_type=jnp.float32)
        kn = jnp.sqrt(jnp.sum(kb * kb, axis=1, keepdims=True))
        qn = jnp.sqrt(jnp.sum(q * q))
        sims = kq[:, 0] / jnp.maximum(kn[:, 0] * qn, 1e-8)
        sims_ref[i, :] = sims.reshape(1, ROWS)[0, :]

    @pl.when(i == NBLK - 1)
    def _topk_gather():
        s = sims_ref[...]
        fidx = (lax.broadcasted_iota(jnp.int32, (NBLK, ROWS), 0) * ROWS
                + lax.broadcasted_iota(jnp.int32, (NBLK, ROWS), 1))
        copies = []
        for k in range(TOPK):
            m = jnp.max(s)
            cand = jnp.where(s == m, fidx, jnp.int32(2 ** 30))
            idx = jnp.min(cand)
            s = jnp.where(fidx == idx, -jnp.inf, s)
            c = pltpu.make_async_copy(
                prompts_hbm.at[pl.ds(idx, 1), :],
                sel_ref.at[pl.ds(k, 1), :],
                gsem,
            )
            c.start()
            copies.append(c)
        for c in copies:
            c.wait()

    @pl.when(i >= NBLK)
    def _add():
        b = i - NBLK
        slot = lax.rem(b, RING)
        pltpu.make_async_copy(
            x_hbm.at[pl.ds(b * BBLK, BBLK)],
            x_bufs.at[slot],
            xsems.at[slot],
        ).wait()
        o_ref[...] = x_bufs[slot] + sel_ref[...][None, :, :]


@jax.jit
def kernel(x, query, prompts, keys):
    q2 = query.reshape(1, EMB)
    out = pl.pallas_call(
        _fused_kernel,
        grid=(NBLK + NB,),
        in_specs=[
            pl.BlockSpec((1, EMB), lambda i: (0, 0)),
            pl.BlockSpec((ROWS, EMB), lambda i: (jnp.minimum(i, NBLK - 1), 0)),
            pl.BlockSpec(memory_space=pl.ANY),
            pl.BlockSpec(memory_space=pl.ANY),
        ],
        out_specs=pl.BlockSpec((BBLK, TOPK, EMB),
                               lambda i: (jnp.maximum(i - NBLK, 0), 0, 0)),
        out_shape=jax.ShapeDtypeStruct((BATCH, TOPK, EMB), jnp.float32),
        scratch_shapes=[
            pltpu.VMEM((NBLK, ROWS), jnp.float32),
            pltpu.VMEM((TOPK, EMB), jnp.float32),
            pltpu.VMEM((RING, BBLK, TOPK, EMB), jnp.float32),
            pltpu.SemaphoreType.DMA,
            pltpu.SemaphoreType.DMA((RING,)),
        ],
    )(q2, keys, x, prompts)
    return out


# speedup vs baseline: 1.3000x; 1.0031x over previous
import jax
import jax.numpy as jnp
from jax import lax
from jax.experimental import pallas as pl
from jax.experimental.pallas import tpu as pltpu

EMB = 1024
NPOOL = 8192
TOPK = 8
NBLK = 8
ROWS = NPOOL // NBLK

BATCH = 4096
BBLK = 128
NB = BATCH // BBLK
RING = NBLK + 2


def _fused_kernel(q_ref, keys_ref, x_hbm, prompts_hbm, o_ref,
                  sims_ref, sel_ref, x_bufs, gsem, xsems):
    i = pl.program_id(0)

    def _issue_block(j):
        slot = lax.rem(j, RING)
        pltpu.make_async_copy(
            x_hbm.at[pl.ds(j * BBLK, BBLK)],
            x_bufs.at[slot],
            xsems.at[slot],
        ).start()

    @pl.when(i == 0)
    def _issue_first():
        _issue_block(jnp.int32(0))
        _issue_block(jnp.int32(1))

    @pl.when((i >= 1) & (i + 1 < NB))
    def _issue_next():
        _issue_block(i + 1)

    @pl.when(i < NBLK)
    def _sims():
        kb = keys_ref[...]
        q = q_ref[...]
        kq = jnp.dot(kb.astype(jnp.bfloat16), q.T.astype(jnp.bfloat16),
                     preferred_element_type=jnp.float32)
        kn = jnp.sqrt(jnp.sum(kb * kb, axis=1, keepdims=True))
        qn = jnp.sqrt(jnp.sum(q * q))
        sims = kq[:, 0] / jnp.maximum(kn[:, 0] * qn, 1e-8)
        sims_ref[i, :] = sims.reshape(1, ROWS)[0, :]

    @pl.when(i == NBLK - 1)
    def _topk_gather():
        s = sims_ref[...]
        fidx = (lax.broadcasted_iota(jnp.int32, (NBLK, ROWS), 0) * ROWS
                + lax.broadcasted_iota(jnp.int32, (NBLK, ROWS), 1))
        copies = []
        for k in range(TOPK):
            m = jnp.max(s)
            cand = jnp.where(s == m, fidx, jnp.int32(2 ** 30))
            idx = jnp.min(cand)
            s = jnp.where(fidx == idx, -jnp.inf, s)
            c = pltpu.make_async_copy(
                prompts_hbm.at[pl.ds(idx, 1), :],
                sel_ref.at[pl.ds(k, 1), :],
                gsem,
            )
            c.start()
            copies.append(c)
        for c in copies:
            c.wait()

    @pl.when(i >= NBLK)
    def _add():
        b = i - NBLK
        slot = lax.rem(b, RING)
        pltpu.make_async_copy(
            x_hbm.at[pl.ds(b * BBLK, BBLK)],
            x_bufs.at[slot],
            xsems.at[slot],
        ).wait()
        o_ref[...] = x_bufs[slot] + sel_ref[...][None, :, :]


@jax.jit
def kernel(x, query, prompts, keys):
    q2 = query.reshape(1, EMB)
    out = pl.pallas_call(
        _fused_kernel,
        grid=(NBLK + NB,),
        in_specs=[
            pl.BlockSpec((1, EMB), lambda i: (0, 0)),
            pl.BlockSpec((ROWS, EMB), lambda i: (jnp.minimum(i, NBLK - 1), 0)),
            pl.BlockSpec(memory_space=pl.ANY),
            pl.BlockSpec(memory_space=pl.ANY),
        ],
        out_specs=pl.BlockSpec((BBLK, TOPK, EMB),
                               lambda i: (jnp.maximum(i - NBLK, 0), 0, 0)),
        out_shape=jax.ShapeDtypeStruct((BATCH, TOPK, EMB), jnp.float32),
        scratch_shapes=[
            pltpu.VMEM((NBLK, ROWS), jnp.float32),
            pltpu.VMEM((TOPK, EMB), jnp.float32),
            pltpu.VMEM((RING, BBLK, TOPK, EMB), jnp.float32),
            pltpu.SemaphoreType.DMA,
            pltpu.SemaphoreType.DMA((RING,)),
        ],
    )(q2, keys, x, prompts)
    return out
